# Initial kernel scaffold; baseline (speedup 1.0000x reference)
import jax
import jax.numpy as jnp
from jax.experimental import pallas as pl


def kernel(x, embed, W1, b1, alpha, W2, b2):
    raise NotImplementedError("write your pallas kernel here")


# speedup vs baseline: 4.8789x; 4.8789x over previous
import functools

import jax
import jax.numpy as jnp
from jax import lax
from jax.experimental import pallas as pl
from jax.experimental.pallas import tpu as pltpu
from jax.experimental.pallas import tpu_sc as plsc

OUT_DIM = 16


def _mlp_table_body(embed_ref, W1_ref, b1_ref, alpha_ref, W2_ref, b2_ref,
                    out_ref):
    e = embed_ref[...]
    h = lax.dot(e, W1_ref[...], preferred_element_type=jnp.float32)
    h = h + b1_ref[...]
    a = alpha_ref[0, 0]
    h = jnp.maximum(h, 0.0) + a * jnp.minimum(h, 0.0)
    out_ref[...] = (lax.dot(h, W2_ref[...], preferred_element_type=jnp.float32)
                    + b2_ref[...])


def _mlp_table(embed, W1, b1, alpha, W2, b2):
    vocab = embed.shape[0]
    return pl.pallas_call(
        _mlp_table_body,
        out_shape=jax.ShapeDtypeStruct((vocab, OUT_DIM), jnp.float32),
        in_specs=[
            pl.BlockSpec(memory_space=pltpu.VMEM),
            pl.BlockSpec(memory_space=pltpu.VMEM),
            pl.BlockSpec(memory_space=pltpu.VMEM),
            pl.BlockSpec(memory_space=pltpu.SMEM),
            pl.BlockSpec(memory_space=pltpu.VMEM),
            pl.BlockSpec(memory_space=pltpu.VMEM),
        ],
        out_specs=pl.BlockSpec(memory_space=pltpu.VMEM),
    )(embed, W1, b1.reshape(1, -1), alpha.reshape(1, 1), W2,
      b2.reshape(1, -1))


def _sc_gather(table, idx4d, nc, ns):
    nw, nchunks, K, lanes = idx4d.shape
    C = K * lanes
    mesh = plsc.VectorSubcoreMesh(core_axis_name="c", subcore_axis_name="s")

    @functools.partial(
        pl.kernel,
        out_type=jax.ShapeDtypeStruct((nw, nchunks, C, OUT_DIM), jnp.float32),
        mesh=mesh,
        scratch_types=[
            pltpu.VMEM((K, lanes), jnp.int32),
            pltpu.VMEM((C, OUT_DIM), jnp.float32),
            pltpu.SemaphoreType.DMA,
        ],
        compiler_params=pltpu.CompilerParams(use_tc_tiling_on_sc=False),
    )
    def gather_kernel(table_hbm, idx_hbm, out_hbm, idx_v, rows_v, sem):
        wid = lax.axis_index("s") * nc + lax.axis_index("c")

        def chunk(j, carry):
            pltpu.sync_copy(idx_hbm.at[wid, j], idx_v)
            handles = [
                pltpu.async_copy(table_hbm.at[idx_v.at[t]],
                                 rows_v.at[pl.ds(t * lanes, lanes)], sem)
                for t in range(K)
            ]
            for h in handles:
                h.wait()
            pltpu.sync_copy(rows_v, out_hbm.at[wid, j])
            return carry

        lax.fori_loop(0, nchunks, chunk, 0)

    return gather_kernel(table, idx4d)


def kernel(x, embed, W1, b1, alpha, W2, b2):
    B = x.size
    info = plsc.get_sparse_core_info()
    nc, ns = info.num_cores, info.num_subcores
    nw = nc * ns
    lanes = 128
    K = 20
    per_w = B // (nw * lanes)
    nchunks = per_w // K
    assert B == nw * nchunks * K * lanes, (B, nw, nchunks, K)

    table = _mlp_table(embed, W1, b1, alpha, W2, b2)
    idx4d = x.reshape(nw, nchunks, K, lanes)
    out = _sc_gather(table, idx4d, nc, ns)
    return out.reshape(B, OUT_DIM)
